# baseline (speedup 1.0000x reference)
import functools

import jax
import jax.numpy as jnp
from jax import lax
from jax.experimental import pallas as pl
from jax.experimental.pallas import tpu as pltpu
from jax.experimental.pallas import tpu_sc as plsc


@functools.cache
def _make_gather(D, L):
    info = plsc.get_sparse_core_info()
    NC, NS, NL = info.num_cores, info.num_subcores, info.num_lanes
    NW = NC * NS
    DPW = D // NW
    KO = L // 128
    mesh = plsc.VectorSubcoreMesh(core_axis_name="c", subcore_axis_name="s")

    @functools.partial(
        pl.kernel,
        mesh=mesh,
        out_type=jax.ShapeDtypeStruct((D, L), jnp.float32),
        scratch_types=[
            pltpu.VMEM((L,), jnp.int32),
            pltpu.VMEM((DPW, L), jnp.float32),
            pltpu.VMEM((KO, DPW, 128), jnp.float32),
            pltpu.SemaphoreType.DMA,
            pltpu.SemaphoreType.DMA,
            pltpu.SemaphoreType.DMA,
        ],
        compiler_params=pltpu.CompilerParams(needs_layout_passes=False),
    )
    def gather_kernel(tab_hbm, idx_hbm, out_hbm, idx_v, slab_v, obuf_v,
                      isem, gsem, wsem):
        wid = lax.axis_index("s") * NC + lax.axis_index("c")
        icopy = pltpu.make_async_copy(idx_hbm.at[0], idx_v, isem)
        icopy.start()
        scopy = pltpu.make_async_copy(
            tab_hbm.at[pl.ds(DPW * wid, DPW), :], slab_v, gsem)
        scopy.start()
        icopy.wait()
        scopy.wait()

        dsplat = [jnp.full((NL,), d, jnp.int32) for d in range(DPW)]

        def body(k, _):
            for j in range(8):
                v = k * 8 + j
                ivec = idx_v[pl.ds(v * NL, NL)]
                for d in range(DPW):
                    val = plsc.load_gather(slab_v, [dsplat[d], ivec])
                    obuf_v[k, d, pl.ds(j * NL, NL)] = val
            pltpu.make_async_copy(
                obuf_v.at[k],
                out_hbm.at[pl.ds(DPW * wid, DPW), pl.ds(128 * k, 128)],
                wsem,
            ).start()
            return _

        lax.fori_loop(0, KO, body, 0)

        for k in range(KO):
            pltpu.make_async_copy(
                obuf_v.at[k],
                out_hbm.at[pl.ds(DPW * wid, DPW), pl.ds(128 * k, 128)],
                wsem,
            ).wait()

    return gather_kernel


def kernel(x, device, table, pe):
    L = x.shape[1]
    D = table.shape[1]
    out_t = _make_gather(D, L)(table.T, pe)
    return out_t.T.reshape(1, L, D)


# speedup vs baseline: 1.0383x; 1.0383x over previous
import functools

import jax
import jax.numpy as jnp
from jax import lax
from jax.experimental import pallas as pl
from jax.experimental.pallas import tpu as pltpu
from jax.experimental.pallas import tpu_sc as plsc


@functools.cache
def _make_gather(D, L):
    info = plsc.get_sparse_core_info()
    NC, NS, NL = info.num_cores, info.num_subcores, info.num_lanes
    NW = NC * NS
    SLABS = D // 8
    QW = NW // SLABS
    CPW = L // QW
    KT = L // 128
    KO = CPW // 128
    NV = CPW // NL
    mesh = plsc.VectorSubcoreMesh(core_axis_name="c", subcore_axis_name="s")

    @functools.partial(
        pl.kernel,
        mesh=mesh,
        out_type=jax.ShapeDtypeStruct((D, L), jnp.float32),
        scratch_types=[
            pltpu.VMEM((CPW,), jnp.int32),
            pltpu.VMEM((8, L), jnp.float32),
            pltpu.VMEM((KO, 8, 128), jnp.float32),
            pltpu.SemaphoreType.DMA,
            pltpu.SemaphoreType.DMA,
            pltpu.SemaphoreType.DMA,
        ],
        compiler_params=pltpu.CompilerParams(
            needs_layout_passes=False,
            disable_bounds_checks=True,
            disable_semaphore_checks=True,
        ),
    )
    def gather_kernel(tab_hbm, idx_hbm, out_hbm, idx_v, slab_v, obuf_v,
                      isem, gsem, wsem):
        wid = lax.axis_index("s") * NC + lax.axis_index("c")
        s = wid // QW
        q = wid % QW
        icopy = pltpu.make_async_copy(
            idx_hbm.at[0, pl.ds(q * CPW, CPW)], idx_v, isem)
        icopy.start()
        scopy = pltpu.make_async_copy(
            tab_hbm.at[pl.ds(8 * s, 8), :],
            slab_v,
            gsem,
        )
        scopy.start()
        icopy.wait()
        scopy.wait()

        dsplat = [jnp.full((NL,), d, jnp.int32) for d in range(8)]

        def body(k, _):
            for j in range(8):
                v = k * 8 + j
                ivec = idx_v[pl.ds(v * NL, NL)]
                for d in range(8):
                    val = plsc.load_gather(slab_v, [dsplat[d], ivec])
                    obuf_v[k, d, pl.ds(j * NL, NL)] = val
            pltpu.make_async_copy(
                obuf_v.at[k],
                out_hbm.at[pl.ds(8 * s, 8),
                           pl.ds(128 * (KO * q + k), 128)],
                wsem,
            ).start()
            return _

        lax.fori_loop(0, KO, body, 0)

        for k in range(KO):
            pltpu.make_async_copy(
                obuf_v.at[k],
                out_hbm.at[pl.ds(8 * s, 8), pl.ds(128 * (KO * q + k), 128)],
                wsem,
            ).wait()

    return gather_kernel


def kernel(x, device, table, pe):
    L = x.shape[1]
    D = table.shape[1]
    out_t = _make_gather(D, L)(table.T, pe)
    return out_t.T.reshape(1, L, D)


# speedup vs baseline: 1.0407x; 1.0023x over previous
import functools

import jax
import jax.numpy as jnp
from jax import lax
from jax.experimental import pallas as pl
from jax.experimental.pallas import tpu as pltpu
from jax.experimental.pallas import tpu_sc as plsc


@functools.cache
def _make_gather(D, L):
    info = plsc.get_sparse_core_info()
    NC, NS, NL = info.num_cores, info.num_subcores, info.num_lanes
    NW = NC * NS
    SLABS = D // 8
    QW = NW // SLABS
    CPW = L // QW
    KT = L // 128
    KO = CPW // 128
    NV = CPW // NL
    mesh = plsc.VectorSubcoreMesh(core_axis_name="c", subcore_axis_name="s")

    @functools.partial(
        pl.kernel,
        mesh=mesh,
        out_type=jax.ShapeDtypeStruct((D, L), jnp.float32),
        scratch_types=[
            pltpu.VMEM((CPW,), jnp.int32),
            pltpu.VMEM((8, L), jnp.float32),
            pltpu.VMEM((KO, 8, 128), jnp.float32),
            pltpu.SemaphoreType.DMA,
            pltpu.SemaphoreType.DMA,
            pltpu.SemaphoreType.DMA,
        ],
        compiler_params=pltpu.CompilerParams(
            needs_layout_passes=False,
            disable_bounds_checks=True,
            disable_semaphore_checks=True,
            skip_device_barrier=True,
        ),
    )
    def gather_kernel(tab_hbm, idx_hbm, out_hbm, idx_v, slab_v, obuf_v,
                      isem, gsem, wsem):
        wid = lax.axis_index("s") * NC + lax.axis_index("c")
        s = wid // QW
        q = wid % QW
        icopy = pltpu.make_async_copy(
            idx_hbm.at[0, pl.ds(q * CPW, CPW)], idx_v, isem)
        icopy.start()
        scopy = pltpu.make_async_copy(
            tab_hbm.at[pl.ds(8 * s, 8), :],
            slab_v,
            gsem,
        )
        scopy.start()
        icopy.wait()
        scopy.wait()

        dsplat = [jnp.full((NL,), d, jnp.int32) for d in range(8)]

        def body(k, _):
            for j in range(8):
                v = k * 8 + j
                ivec = idx_v[pl.ds(v * NL, NL)]
                for d in range(8):
                    val = plsc.load_gather(slab_v, [dsplat[d], ivec])
                    obuf_v[k, d, pl.ds(j * NL, NL)] = val
            pltpu.make_async_copy(
                obuf_v.at[k],
                out_hbm.at[pl.ds(8 * s, 8),
                           pl.ds(128 * (KO * q + k), 128)],
                wsem,
            ).start()
            return _

        lax.fori_loop(0, KO, body, 0)

        for k in range(KO):
            pltpu.make_async_copy(
                obuf_v.at[k],
                out_hbm.at[pl.ds(8 * s, 8), pl.ds(128 * (KO * q + k), 128)],
                wsem,
            ).wait()

    return gather_kernel


def kernel(x, device, table, pe):
    L = x.shape[1]
    D = table.shape[1]
    out_t = _make_gather(D, L)(table.T, pe)
    return out_t.T.reshape(1, L, D)
